# baseline (speedup 1.0000x reference)
import jax
import jax.numpy as jnp
from jax.experimental import pallas as pl
from jax.experimental.pallas import tpu as pltpu

_B, _D, _K = 128, 8732, 21


def _body(x_ref, o_ref, acc_ref):
    i = pl.program_id(0)
    j = pl.program_id(1)

    @pl.when((i == 0) & (j == 0))
    def _init():
        acc_ref[0] = 0.0

    acc_ref[0] += jnp.sum(x_ref[...])

    @pl.when((i == 15) & (j == 68))
    def _fin():
        o_ref[...] = jnp.full((1, 1), acc_ref[0], jnp.float32)


def kernel(loc_preds, loc_targets, conf_preds, conf_targets):
    x2 = conf_preds.reshape(_B, _D * _K)
    o = pl.pallas_call(
        _body,
        grid=(16, 69),
        in_specs=[pl.BlockSpec((8, 2688), lambda i, j: (i, j))],
        out_specs=pl.BlockSpec((1, 1), lambda i, j: (0, 0)),
        out_shape=jax.ShapeDtypeStruct((1, 1), jnp.float32),
        scratch_shapes=[pltpu.SMEM((1,), jnp.float32)],
    )(x2)
    return (o[0, 0], o[0, 0], o[0, 0])


# speedup vs baseline: 1.3949x; 1.3949x over previous
import jax
import jax.numpy as jnp
from jax import lax
from jax.experimental import pallas as pl
from jax.experimental.pallas import tpu as pltpu

_K = 21
_B = 128
_D = 8732
_NC = 4
_CB = _B // _NC
_MAX_FINITE_BITS = 0x7F800000


def _loc_body(lp_ref, lt_ref, y_ref, npos_ref, o_ref, acc_ref):
    i = pl.program_id(0)

    @pl.when(i == 0)
    def _init():
        acc_ref[0] = 0.0

    y = y_ref[...]
    pos = y > 0
    posf = pos.astype(jnp.float32)
    npos_ref[...] = jnp.sum(posf, axis=1, keepdims=True)

    z = jnp.abs(lp_ref[...] - lt_ref[...])
    sl1 = jnp.where(z < 1.0, 0.5 * z * z, z - 0.5)
    acc_ref[0] += jnp.sum(jnp.where(pos[None], sl1, 0.0))

    @pl.when(i == _B // 8 - 1)
    def _fin():
        o_ref[...] = jnp.full((1, 1), acc_ref[0], jnp.float32)


def _conf_body(x_ref, y_ref, cl_ref, o_ref, acc_ref):
    i = pl.program_id(0)

    @pl.when(i == 0)
    def _init():
        acc_ref[0] = 0.0

    x = x_ref[...]
    y = y_ref[...]
    m = jnp.max(x, axis=0)
    s = jnp.sum(jnp.exp(x - m[None]), axis=0)
    lse = jnp.log(s) + m
    ks = lax.broadcasted_iota(jnp.int32, (_K, 8, _D), 0)
    g = jnp.sum(jnp.where(y[None] == ks, x, 0.0), axis=0)
    ce = lse - g

    pos = y > 0
    acc_ref[0] += jnp.sum(jnp.where(pos, ce, 0.0))
    cl_ref[...] = jnp.where(pos, 0.0, ce)

    @pl.when(i == _CB // 8 - 1)
    def _fin():
        o_ref[...] = jnp.full((1, 1), acc_ref[0], jnp.float32)


def _fin_body(cl0_ref, cl1_ref, cl2_ref, cl3_ref, npos_ref, loc_ref,
              ce0_ref, ce1_ref, ce2_ref, ce3_ref,
              o0_ref, o1_ref, o2_ref):
    cl_refs = (cl0_ref, cl1_ref, cl2_ref, cl3_ref)
    bits = [lax.bitcast_convert_type(r[...], jnp.int32) for r in cl_refs]
    npos = npos_ref[...]
    k_all = jnp.minimum(3.0 * npos, float(_D - 1))
    ks = [lax.slice(k_all, (c * _CB, 0), ((c + 1) * _CB, 1))
          for c in range(_NC)]

    def step(_, carry):
        out = []
        for c in range(_NC):
            lo, hi = carry[c]
            mid = lo + lax.shift_right_logical(hi - lo + 1, 1)
            cnt = jnp.sum((bits[c] >= mid).astype(jnp.float32), axis=1,
                          keepdims=True)
            ok = cnt >= ks[c]
            out.append((jnp.where(ok, mid, lo), jnp.where(ok, hi, mid - 1)))
        return tuple(out)

    init = tuple((jnp.zeros((_CB, 1), jnp.int32),
                  jnp.full((_CB, 1), _MAX_FINITE_BITS, jnp.int32))
                 for _ in range(_NC))
    final = lax.fori_loop(0, 31, step, init)

    conf_sum = 0.0
    for c in range(_NC):
        v_bits = final[c][0]
        above = bits[c] > v_bits
        n_above = jnp.sum(above.astype(jnp.float32), axis=1, keepdims=True)
        sum_above = jnp.sum(jnp.where(above, cl_refs[c][...], 0.0), axis=1,
                            keepdims=True)
        t = lax.bitcast_convert_type(v_bits, jnp.float32)
        tie = ks[c] - n_above
        conf_sum += jnp.sum(sum_above + jnp.where(tie > 0.0, tie * t, 0.0))

    num_matched = jnp.sum(npos)
    ce_pos = (ce0_ref[0, 0] + ce1_ref[0, 0] + ce2_ref[0, 0] + ce3_ref[0, 0])
    conf_loss = (ce_pos + conf_sum) / num_matched
    loc_loss = loc_ref[0, 0] / num_matched
    o0_ref[...] = jnp.full((1, 1), loc_loss + conf_loss, jnp.float32)
    o1_ref[...] = jnp.full((1, 1), conf_loss, jnp.float32)
    o2_ref[...] = jnp.full((1, 1), loc_loss, jnp.float32)


def kernel(loc_preds, loc_targets, conf_preds, conf_targets):
    lpt = jnp.transpose(loc_preds, (2, 0, 1))
    ltt = jnp.transpose(loc_targets, (2, 0, 1))

    npos, loc_sum = pl.pallas_call(
        _loc_body,
        grid=(_B // 8,),
        in_specs=[
            pl.BlockSpec((4, 8, _D), lambda i: (0, i, 0)),
            pl.BlockSpec((4, 8, _D), lambda i: (0, i, 0)),
            pl.BlockSpec((8, _D), lambda i: (i, 0)),
        ],
        out_specs=[
            pl.BlockSpec((8, 1), lambda i: (i, 0)),
            pl.BlockSpec((1, 1), lambda i: (0, 0)),
        ],
        out_shape=[
            jax.ShapeDtypeStruct((_B, 1), jnp.float32),
            jax.ShapeDtypeStruct((1, 1), jnp.float32),
        ],
        scratch_shapes=[pltpu.SMEM((1,), jnp.float32)],
    )(lpt, ltt, conf_targets)

    cls = []
    ces = []
    for c in range(_NC):
        xt_c = jnp.transpose(conf_preds[c * _CB:(c + 1) * _CB], (2, 0, 1))
        cl_c, ce_c = pl.pallas_call(
            _conf_body,
            grid=(_CB // 8,),
            in_specs=[
                pl.BlockSpec((_K, 8, _D), lambda i: (0, i, 0)),
                pl.BlockSpec((8, _D), lambda i, c=c: (c * _CB // 8 + i, 0)),
            ],
            out_specs=[
                pl.BlockSpec((8, _D), lambda i: (i, 0)),
                pl.BlockSpec((1, 1), lambda i: (0, 0)),
            ],
            out_shape=[
                jax.ShapeDtypeStruct((_CB, _D), jnp.float32),
                jax.ShapeDtypeStruct((1, 1), jnp.float32),
            ],
            scratch_shapes=[pltpu.SMEM((1,), jnp.float32)],
        )(xt_c, conf_targets)
        cls.append(cl_c)
        ces.append(ce_c)

    full = lambda: (0, 0)
    o0, o1, o2 = pl.pallas_call(
        _fin_body,
        grid=(1,),
        in_specs=[pl.BlockSpec((_CB, _D), lambda i: (0, 0))] * _NC
        + [pl.BlockSpec((_B, 1), lambda i: (0, 0))]
        + [pl.BlockSpec((1, 1), lambda i: (0, 0))] * 5,
        out_specs=[pl.BlockSpec((1, 1), lambda i: (0, 0))] * 3,
        out_shape=[jax.ShapeDtypeStruct((1, 1), jnp.float32)] * 3,
    )(cls[0], cls[1], cls[2], cls[3], npos, loc_sum,
      ces[0], ces[1], ces[2], ces[3])
    return (o0[0, 0], o1[0, 0], o2[0, 0])


# speedup vs baseline: 1.3950x; 1.0001x over previous
import jax
import jax.numpy as jnp
from jax import lax
from jax.experimental import pallas as pl
from jax.experimental.pallas import tpu as pltpu

_K = 21
_B = 128
_D = 8732
_NC = 4
_CB = _B // _NC
_MAX_FINITE_BITS = 0x7F800000


def _loc_body(lp_ref, lt_ref, y_ref, npos_ref, o_ref, acc_ref):
    i = pl.program_id(0)

    @pl.when(i == 0)
    def _init():
        acc_ref[0] = 0.0

    y = y_ref[...]
    pos = y > 0
    posf = pos.astype(jnp.float32)
    npos_ref[...] = jnp.sum(posf, axis=1, keepdims=True)

    z = jnp.abs(lp_ref[...] - lt_ref[...])
    sl1 = jnp.where(z < 1.0, 0.5 * z * z, z - 0.5)
    acc_ref[0] += jnp.sum(jnp.where(pos[None], sl1, 0.0))

    @pl.when(i == _B // 8 - 1)
    def _fin():
        o_ref[...] = jnp.full((1, 1), acc_ref[0], jnp.float32)


def _conf_body(x_ref, y_ref, cl_ref, o_ref, acc_ref):
    i = pl.program_id(0)

    @pl.when(i == 0)
    def _init():
        acc_ref[0] = 0.0

    x = x_ref[...]
    y = y_ref[...]
    m = jnp.max(x, axis=0)
    s = jnp.sum(jnp.exp(x - m[None]), axis=0)
    lse = jnp.log(s) + m
    ks = lax.broadcasted_iota(jnp.int32, (_K, 8, _D), 0)
    g = jnp.sum(jnp.where(y[None] == ks, x, 0.0), axis=0)
    ce = lse - g

    pos = y > 0
    acc_ref[0] += jnp.sum(jnp.where(pos, ce, 0.0))
    cl_ref[...] = jnp.where(pos, 0.0, ce)

    @pl.when(i == _CB // 8 - 1)
    def _fin():
        o_ref[...] = jnp.full((1, 1), acc_ref[0], jnp.float32)


def _fin_body(cl0_ref, cl1_ref, cl2_ref, cl3_ref, npos_ref, loc_ref,
              ce0_ref, ce1_ref, ce2_ref, ce3_ref,
              o0_ref, o1_ref, o2_ref):
    cl_refs = (cl0_ref, cl1_ref, cl2_ref, cl3_ref)
    bits = [lax.bitcast_convert_type(r[...], jnp.int32) for r in cl_refs]
    npos = npos_ref[...]
    k_all = jnp.minimum(3.0 * npos, float(_D - 1))
    ks = [lax.slice(k_all, (c * _CB, 0), ((c + 1) * _CB, 1))
          for c in range(_NC)]

    def step(_, carry):
        out = []
        for c in range(_NC):
            lo, hi = carry[c]
            mid = lo + lax.shift_right_logical(hi - lo + 1, 1)
            cnt = jnp.sum((bits[c] >= mid).astype(jnp.float32), axis=1,
                          keepdims=True)
            ok = cnt >= ks[c]
            out.append((jnp.where(ok, mid, lo), jnp.where(ok, hi, mid - 1)))
        return tuple(out)

    init = tuple((jnp.zeros((_CB, 1), jnp.int32),
                  jnp.full((_CB, 1), _MAX_FINITE_BITS, jnp.int32))
                 for _ in range(_NC))
    final = lax.fori_loop(0, 31, step, init)

    conf_sum = 0.0
    for c in range(_NC):
        v_bits = final[c][0]
        above = bits[c] > v_bits
        n_above = jnp.sum(above.astype(jnp.float32), axis=1, keepdims=True)
        sum_above = jnp.sum(jnp.where(above, cl_refs[c][...], 0.0), axis=1,
                            keepdims=True)
        t = lax.bitcast_convert_type(v_bits, jnp.float32)
        tie = ks[c] - n_above
        conf_sum += jnp.sum(sum_above + jnp.where(tie > 0.0, tie * t, 0.0))

    num_matched = jnp.sum(npos)
    ce_pos = (ce0_ref[0, 0] + ce1_ref[0, 0] + ce2_ref[0, 0] + ce3_ref[0, 0])
    conf_loss = (ce_pos + conf_sum) / num_matched
    loc_loss = loc_ref[0, 0] / num_matched
    o0_ref[...] = jnp.full((1, 1), loc_loss + conf_loss, jnp.float32)
    o1_ref[...] = jnp.full((1, 1), conf_loss, jnp.float32)
    o2_ref[...] = jnp.full((1, 1), loc_loss, jnp.float32)


def kernel(loc_preds, loc_targets, conf_preds, conf_targets):
    lpt = jnp.transpose(loc_preds, (2, 0, 1))
    ltt = jnp.transpose(loc_targets, (2, 0, 1))

    npos, loc_sum = pl.pallas_call(
        _loc_body,
        grid=(_B // 8,),
        in_specs=[
            pl.BlockSpec((4, 8, _D), lambda i: (0, i, 0)),
            pl.BlockSpec((4, 8, _D), lambda i: (0, i, 0)),
            pl.BlockSpec((8, _D), lambda i: (i, 0)),
        ],
        out_specs=[
            pl.BlockSpec((8, 1), lambda i: (i, 0)),
            pl.BlockSpec((1, 1), lambda i: (0, 0)),
        ],
        out_shape=[
            jax.ShapeDtypeStruct((_B, 1), jnp.float32),
            jax.ShapeDtypeStruct((1, 1), jnp.float32),
        ],
        scratch_shapes=[pltpu.SMEM((1,), jnp.float32)],
    )(lpt, ltt, conf_targets)

    cls = []
    ces = []
    for c in range(_NC):
        part = lax.optimization_barrier(conf_preds[c * _CB:(c + 1) * _CB])
        xt_c = jnp.transpose(part, (2, 0, 1))
        cl_c, ce_c = pl.pallas_call(
            _conf_body,
            grid=(_CB // 8,),
            in_specs=[
                pl.BlockSpec((_K, 8, _D), lambda i: (0, i, 0)),
                pl.BlockSpec((8, _D), lambda i, c=c: (c * _CB // 8 + i, 0)),
            ],
            out_specs=[
                pl.BlockSpec((8, _D), lambda i: (i, 0)),
                pl.BlockSpec((1, 1), lambda i: (0, 0)),
            ],
            out_shape=[
                jax.ShapeDtypeStruct((_CB, _D), jnp.float32),
                jax.ShapeDtypeStruct((1, 1), jnp.float32),
            ],
            scratch_shapes=[pltpu.SMEM((1,), jnp.float32)],
        )(xt_c, conf_targets)
        cls.append(cl_c)
        ces.append(ce_c)

    full = lambda: (0, 0)
    o0, o1, o2 = pl.pallas_call(
        _fin_body,
        grid=(1,),
        in_specs=[pl.BlockSpec((_CB, _D), lambda i: (0, 0))] * _NC
        + [pl.BlockSpec((_B, 1), lambda i: (0, 0))]
        + [pl.BlockSpec((1, 1), lambda i: (0, 0))] * 5,
        out_specs=[pl.BlockSpec((1, 1), lambda i: (0, 0))] * 3,
        out_shape=[jax.ShapeDtypeStruct((1, 1), jnp.float32)] * 3,
    )(cls[0], cls[1], cls[2], cls[3], npos, loc_sum,
      ces[0], ces[1], ces[2], ces[3])
    return (o0[0, 0], o1[0, 0], o2[0, 0])


# speedup vs baseline: 3.6508x; 2.6171x over previous
import jax
import jax.numpy as jnp
from jax import lax
from jax.experimental import pallas as pl
from jax.experimental.pallas import tpu as pltpu

_K = 21
_B = 128
_D = 8732
_NC = 4
_CB = _B // _NC
_MAX_FINITE_BITS = 0x7F800000


def _loc_body(lp_ref, lt_ref, y_ref, npos_ref, o_ref, acc_ref):
    i = pl.program_id(0)

    @pl.when(i == 0)
    def _init():
        acc_ref[0] = 0.0

    y = y_ref[...]
    pos = y > 0
    posf = pos.astype(jnp.float32)
    npos_ref[...] = jnp.sum(posf, axis=1, keepdims=True)

    z = jnp.abs(lp_ref[...] - lt_ref[...])
    sl1 = jnp.where(z < 1.0, 0.5 * z * z, z - 0.5)
    acc_ref[0] += jnp.sum(jnp.where(pos[None], sl1, 0.0))

    @pl.when(i == _B // 8 - 1)
    def _fin():
        o_ref[...] = jnp.full((1, 1), acc_ref[0], jnp.float32)


def _conf_body(x_ref, y_ref, cl_ref, o_ref, acc_ref):
    i = pl.program_id(0)

    @pl.when(i == 0)
    def _init():
        acc_ref[0] = 0.0

    x = x_ref[...]
    y = y_ref[...]
    m = jnp.max(x, axis=0)
    s = jnp.sum(jnp.exp(x - m[None]), axis=0)
    lse = jnp.log(s) + m
    ks = lax.broadcasted_iota(jnp.int32, (_K, 8, _D), 0)
    g = jnp.sum(jnp.where(y[None] == ks, x, 0.0), axis=0)
    ce = lse - g

    pos = y > 0
    acc_ref[0] += jnp.sum(jnp.where(pos, ce, 0.0))
    cl_ref[...] = jnp.where(pos, 0.0, ce)

    @pl.when(i == _CB // 8 - 1)
    def _fin():
        o_ref[...] = jnp.full((1, 1), acc_ref[0], jnp.float32)


def _fin_body(cl0_ref, cl1_ref, cl2_ref, cl3_ref, npos_ref, loc_ref,
              ce0_ref, ce1_ref, ce2_ref, ce3_ref,
              o0_ref, o1_ref, o2_ref):
    cl_refs = (cl0_ref, cl1_ref, cl2_ref, cl3_ref)
    bits = [lax.bitcast_convert_type(r[...], jnp.int32) for r in cl_refs]
    npos = npos_ref[...]
    k_all = jnp.minimum(3.0 * npos, float(_D - 1))
    ks = [lax.slice(k_all, (c * _CB, 0), ((c + 1) * _CB, 1))
          for c in range(_NC)]

    def step(_, carry):
        out = []
        for c in range(_NC):
            lo, hi = carry[c]
            mid = lo + lax.shift_right_logical(hi - lo + 1, 1)
            cnt = jnp.sum((bits[c] >= mid).astype(jnp.float32), axis=1,
                          keepdims=True)
            ok = cnt >= ks[c]
            out.append((jnp.where(ok, mid, lo), jnp.where(ok, hi, mid - 1)))
        return tuple(out)

    init = tuple((jnp.zeros((_CB, 1), jnp.int32),
                  jnp.full((_CB, 1), _MAX_FINITE_BITS, jnp.int32))
                 for _ in range(_NC))
    final = lax.fori_loop(0, 31, step, init)

    conf_sum = 0.0
    for c in range(_NC):
        v_bits = final[c][0]
        above = bits[c] > v_bits
        n_above = jnp.sum(above.astype(jnp.float32), axis=1, keepdims=True)
        sum_above = jnp.sum(jnp.where(above, cl_refs[c][...], 0.0), axis=1,
                            keepdims=True)
        t = lax.bitcast_convert_type(v_bits, jnp.float32)
        tie = ks[c] - n_above
        conf_sum += jnp.sum(sum_above + jnp.where(tie > 0.0, tie * t, 0.0))

    num_matched = jnp.sum(npos)
    ce_pos = (ce0_ref[0, 0] + ce1_ref[0, 0] + ce2_ref[0, 0] + ce3_ref[0, 0])
    conf_loss = (ce_pos + conf_sum) / num_matched
    loc_loss = loc_ref[0, 0] / num_matched
    o0_ref[...] = jnp.full((1, 1), loc_loss + conf_loss, jnp.float32)
    o1_ref[...] = jnp.full((1, 1), conf_loss, jnp.float32)
    o2_ref[...] = jnp.full((1, 1), loc_loss, jnp.float32)


def kernel(loc_preds, loc_targets, conf_preds, conf_targets):
    lpt = jnp.transpose(loc_preds, (2, 0, 1))
    ltt = jnp.transpose(loc_targets, (2, 0, 1))

    npos, loc_sum = pl.pallas_call(
        _loc_body,
        grid=(_B // 8,),
        in_specs=[
            pl.BlockSpec((4, 8, _D), lambda i: (0, i, 0)),
            pl.BlockSpec((4, 8, _D), lambda i: (0, i, 0)),
            pl.BlockSpec((8, _D), lambda i: (i, 0)),
        ],
        out_specs=[
            pl.BlockSpec((8, 1), lambda i: (i, 0)),
            pl.BlockSpec((1, 1), lambda i: (0, 0)),
        ],
        out_shape=[
            jax.ShapeDtypeStruct((_B, 1), jnp.float32),
            jax.ShapeDtypeStruct((1, 1), jnp.float32),
        ],
        scratch_shapes=[pltpu.SMEM((1,), jnp.float32)],
    )(lpt, ltt, conf_targets)

    xt = jnp.transpose(conf_preds, (2, 0, 1))
    cls = []
    ces = []
    for c in range(_NC):
        cl_c, ce_c = pl.pallas_call(
            _conf_body,
            grid=(_CB // 8,),
            in_specs=[
                pl.BlockSpec((_K, 8, _D), lambda i, c=c: (0, c * _CB // 8 + i, 0)),
                pl.BlockSpec((8, _D), lambda i, c=c: (c * _CB // 8 + i, 0)),
            ],
            out_specs=[
                pl.BlockSpec((8, _D), lambda i: (i, 0)),
                pl.BlockSpec((1, 1), lambda i: (0, 0)),
            ],
            out_shape=[
                jax.ShapeDtypeStruct((_CB, _D), jnp.float32),
                jax.ShapeDtypeStruct((1, 1), jnp.float32),
            ],
            scratch_shapes=[pltpu.SMEM((1,), jnp.float32)],
        )(xt, conf_targets)
        cls.append(cl_c)
        ces.append(ce_c)

    full = lambda: (0, 0)
    o0, o1, o2 = pl.pallas_call(
        _fin_body,
        grid=(1,),
        in_specs=[pl.BlockSpec((_CB, _D), lambda i: (0, 0))] * _NC
        + [pl.BlockSpec((_B, 1), lambda i: (0, 0))]
        + [pl.BlockSpec((1, 1), lambda i: (0, 0))] * 5,
        out_specs=[pl.BlockSpec((1, 1), lambda i: (0, 0))] * 3,
        out_shape=[jax.ShapeDtypeStruct((1, 1), jnp.float32)] * 3,
    )(cls[0], cls[1], cls[2], cls[3], npos, loc_sum,
      ces[0], ces[1], ces[2], ces[3])
    return (o0[0, 0], o1[0, 0], o2[0, 0])
